# baseline (speedup 1.0000x reference)
import math

import jax
import jax.numpy as jnp
from jax import lax
from jax.experimental import pallas as pl
from jax.experimental.pallas import tpu as pltpu
from jax.experimental.pallas import tpu_sc as plsc

VOCAB = 32000
N_TOK = 2048
PAD = 0
SMOOTHING = 0.1
CONF = 1.0 - SMOOTHING
EPS = SMOOTHING / (VOCAB - 2)
ROW_CONST = (VOCAB - 2) * EPS * math.log(EPS) + CONF * math.log(CONF)

NW = 32
SC_ROWS = 256
TC_ROWS = N_TOK - SC_ROWS
RPW = SC_ROWS // NW
LANES = 16
NCHUNK = VOCAB // LANES

RB = 128
CB = VOCAB


def _loss_kernel(tgt_ref, x_ref, out_ref):
    i = pl.program_id(0)

    @pl.when(i == 0)
    def _():
        out_ref[...] = jnp.zeros((1, 1), jnp.float32)

    x = x_ref[...]
    tgt = tgt_ref[...]
    valid = tgt != PAD

    rowsum = jnp.sum(x, axis=1, keepdims=True)
    cols = jax.lax.broadcasted_iota(jnp.int32, (RB, CB), 1)
    hit = cols == tgt
    xt = jnp.sum(jnp.where(hit, x, 0.0), axis=1, keepdims=True)

    contrib = ROW_CONST - EPS * rowsum + EPS * x[:, 0:1] - (CONF - EPS) * xt
    contrib = jnp.where(valid, contrib, 0.0)
    out_ref[...] += jnp.sum(contrib, axis=0, keepdims=True)


def _sc_body(x_hbm, tgtb_hbm, out_hbm, tgtb_v, buf0, buf1, acc_v, sem0, sem1):
    wid = lax.axis_index("s") * 2 + lax.axis_index("c")
    row0 = TC_ROWS + wid * RPW
    pltpu.sync_copy(tgtb_hbm.at[pl.ds(wid * RPW * LANES, RPW * LANES)], tgtb_v)
    lane_iota = lax.iota(jnp.int32, LANES)

    bufs = (buf0, buf1)
    sems = (sem0, sem1)
    copies = [None, None]
    copies[0] = pltpu.async_copy(x_hbm.at[row0], buf0, sem0)

    acc = jnp.zeros((LANES,), jnp.float32)
    for r in range(RPW):
        if r + 1 < RPW:
            copies[(r + 1) % 2] = pltpu.async_copy(
                x_hbm.at[row0 + r + 1], bufs[(r + 1) % 2], sems[(r + 1) % 2])
        copies[r % 2].wait()
        rowbuf = bufs[r % 2]
        t16 = tgtb_v[pl.ds(r * LANES, LANES)]

        def chunk_step(k, carry):
            lanesum, xtacc = carry
            c = rowbuf[pl.ds(k * LANES, LANES)]
            lanes = k * LANES + lane_iota
            return (lanesum + c, xtacc + jnp.where(lanes == t16, c, 0.0))

        zeros = jnp.zeros((LANES,), jnp.float32)
        lanesum, xtacc = lax.fori_loop(0, NCHUNK, chunk_step, (zeros, zeros),
                                       unroll=16)
        chunk0 = rowbuf[pl.ds(0, LANES)]
        x0vec = jnp.where(lane_iota == 0, chunk0, 0.0)
        m16 = t16 != PAD
        acc = acc + jnp.where(
            m16,
            (ROW_CONST / LANES) - EPS * lanesum + EPS * x0vec
            - (CONF - EPS) * xtacc,
            0.0)

    acc_v[...] = acc
    pltpu.sync_copy(acc_v, out_hbm.at[wid])


def _sc_rows(x, tgtb):
    return pl.kernel(
        _sc_body,
        mesh=plsc.VectorSubcoreMesh(core_axis_name="c", subcore_axis_name="s"),
        out_type=jax.ShapeDtypeStruct((NW, LANES), jnp.float32),
        scratch_types=[
            pltpu.VMEM((RPW * LANES,), jnp.int32),
            pltpu.VMEM((VOCAB,), jnp.float32),
            pltpu.VMEM((VOCAB,), jnp.float32),
            pltpu.VMEM((LANES,), jnp.float32),
            pltpu.SemaphoreType.DMA,
            pltpu.SemaphoreType.DMA,
        ],
    )(x, tgtb)


@jax.jit
def kernel(x, target):
    tgt = target.astype(jnp.int32)
    tgtb = jnp.broadcast_to(tgt[TC_ROWS:, None],
                            (SC_ROWS, LANES)).reshape(SC_ROWS * LANES)
    partials = _sc_rows(x, tgtb)
    dense = pl.pallas_call(
        _loss_kernel,
        grid=(TC_ROWS // RB,),
        in_specs=[
            pl.BlockSpec((RB, 1), lambda i: (i, 0)),
            pl.BlockSpec((RB, CB), lambda i: (i, 0)),
        ],
        out_specs=pl.BlockSpec((1, 1), lambda i: (0, 0)),
        out_shape=jax.ShapeDtypeStruct((1, 1), jnp.float32),
        compiler_params=pltpu.CompilerParams(
            dimension_semantics=("arbitrary",),
        ),
    )(tgt.reshape(N_TOK, 1), x)
    return dense[0, 0] + jnp.sum(partials)


# speedup vs baseline: 1.2450x; 1.2450x over previous
import math

import jax
import jax.numpy as jnp
from jax.experimental import pallas as pl
from jax.experimental.pallas import tpu as pltpu

VOCAB = 32000
N_TOK = 2048
PAD = 0
SMOOTHING = 0.1
CONF = 1.0 - SMOOTHING
EPS = SMOOTHING / (VOCAB - 2)
ROW_CONST = (VOCAB - 2) * EPS * math.log(EPS) + CONF * math.log(CONF)

RB = 128
CB = VOCAB


def _loss_kernel(tgt_ref, x_ref, out_ref):
    i = pl.program_id(0)

    @pl.when(i == 0)
    def _():
        out_ref[...] = jnp.zeros((1, 1), jnp.float32)

    x = x_ref[...]
    tgt = tgt_ref[...]
    valid = tgt != PAD

    rowsum = jnp.sum(x, axis=1, keepdims=True)
    cols = jax.lax.broadcasted_iota(jnp.int32, (RB, CB), 1)
    hit = cols == tgt
    xt = jnp.sum(jnp.where(hit, x, 0.0), axis=1, keepdims=True)

    contrib = ROW_CONST - EPS * rowsum + EPS * x[:, 0:1] - (CONF - EPS) * xt
    contrib = jnp.where(valid, contrib, 0.0)
    out_ref[...] += jnp.sum(contrib, axis=0, keepdims=True)


@jax.jit
def kernel(x, target):
    tgt = target.astype(jnp.int32).reshape(N_TOK, 1)
    out = pl.pallas_call(
        _loss_kernel,
        grid=(N_TOK // RB,),
        in_specs=[
            pl.BlockSpec((RB, 1), lambda i: (i, 0)),
            pl.BlockSpec((RB, CB), lambda i: (i, 0)),
        ],
        out_specs=pl.BlockSpec((1, 1), lambda i: (0, 0)),
        out_shape=jax.ShapeDtypeStruct((1, 1), jnp.float32),
        compiler_params=pltpu.CompilerParams(
            dimension_semantics=("arbitrary",),
        ),
    )(tgt, x)
    return out[0, 0]
